# baseline (speedup 1.0000x reference)
import jax
import jax.numpy as jnp
from jax import lax
from jax.experimental import pallas as pl
from jax.experimental.pallas import tpu as pltpu
from jax.experimental.pallas import tpu_sc as plsc

N = 10000
E = 320000
D = 128
ED = 16

NT = 32
CH = 128
EPT = 10240
E_PAD = NT * EPT
NCH = EPT // CH
N_PAD = 10240
NXR = N_PAD // 32

C_SLOW = 1
CG2 = 256
NCHG = EPT // CG2
KG_SLOW = 26

EB = 512
NE_BLK = E_PAD // EB
NB = 400
NBP = 512

_F32 = jnp.float32


def _pre_body(h_ref, wa_ref, wb_ref, be1_ref, a_ref, b_ref):
    hb = h_ref[...]
    a_ref[...] = jnp.dot(hb, wa_ref[...], preferred_element_type=_F32) \
        + be1_ref[...]
    b_ref[...] = jnp.dot(hb, wb_ref[...], preferred_element_type=_F32)


def _build_tables(h_pad, wa, wb, be1):
    return pl.pallas_call(
        _pre_body,
        grid=(N_PAD // NBP,),
        in_specs=[
            pl.BlockSpec((NBP, D), lambda i: (i, 0)),
            pl.BlockSpec((D, D), lambda i: (0, 0)),
            pl.BlockSpec((D, D), lambda i: (0, 0)),
            pl.BlockSpec((1, D), lambda i: (0, 0)),
        ],
        out_specs=[
            pl.BlockSpec((NBP, D), lambda i: (i, 0)),
            pl.BlockSpec((NBP, D), lambda i: (i, 0)),
        ],
        out_shape=[
            jax.ShapeDtypeStruct((N_PAD, D), _F32),
            jax.ShapeDtypeStruct((N_PAD, D), _F32),
        ],
    )(h_pad, wa, wb, be1)


def _gather_body(a_hbm, b_hbm, xf_hbm, rc_hbm, out_hbm, rad_hbm,
                 xref, idx0, idx1, abuf0, abuf1, rbuf0, rbuf1,
                 gsem0, gsem1, wsem0, wsem1, rsem0, rsem1):
    c = lax.axis_index("c")
    s = lax.axis_index("s")
    wid = s * 2 + c
    n_my = jnp.where(c == C_SLOW, KG_SLOW, 2 * NCHG - KG_SLOW)
    partner = s * 2 + C_SLOW

    def loc(j):
        sel = j < NCHG
        w_j = jnp.where(sel, wid, partner)
        jj = jnp.where(sel, j, j - (NCHG - KG_SLOW))
        return w_j, jj

    pltpu.sync_copy(xf_hbm, xref)
    idxb = (idx0, idx1)
    abufs = (abuf0, abuf1)
    rbufs = (rbuf0, rbuf1)
    gsems = (gsem0, gsem1)
    wsems = (wsem0, wsem1)
    rsems = (rsem0, rsem1)

    def load_idx(j, q):
        wn, jn = loc(j)
        pltpu.sync_copy(rc_hbm.at[wn].at[pl.ds(jn * 2, 2)], idxb[q])

    def issue_a(q):
        for k in (0, 1):
            pltpu.async_copy(a_hbm.at[idxb[q].at[k].at[0]],
                             abufs[q].at[pl.ds(k * CH, CH)], gsems[q])

    load_idx(0, 0)
    issue_a(0)

    def outer(jo, carry):
        j2 = jo * 2
        for p in (0, 1):
            j = j2 + p
            q = 1 - p
            ab = abufs[p]
            rb = rbufs[p]

            @pl.when(j >= 1)
            def _():
                pltpu.make_async_copy(ab, out_hbm.at[pl.ds(0, CG2)],
                                      wsems[q]).wait()
                pltpu.make_async_copy(rb, rad_hbm.at[:, pl.ds(0, CG2)],
                                      rsems[q]).wait()

            @pl.when(j + 1 < n_my)
            def _():
                load_idx(j + 1, q)
                issue_a(q)

            for k in (0, 1):
                pltpu.make_async_copy(a_hbm.at[pl.ds(0, CH)],
                                      ab.at[pl.ds(k * CH, CH)],
                                      gsems[p]).wait()
            for k in (0, 1):
                pltpu.async_copy(b_hbm.at[idxb[p].at[k].at[1]],
                                 ab.at[pl.ds(k * CH, CH)], gsems[p], add=True)

            for k in (0, 1):
                for g in range(CH // 16):
                    sl16 = pl.ds(g * 16, 16)
                    sl16o = pl.ds(k * CH + g * 16, 16)
                    rv4 = idxb[p][k, 0, sl16] * 4
                    cv4 = idxb[p][k, 1, sl16] * 4
                    rad = jnp.zeros((16,), _F32)
                    for j2c in range(3):
                        cdj = plsc.load_gather(xref, [rv4 + j2c]) \
                            - plsc.load_gather(xref, [cv4 + j2c])
                        rb[j2c, sl16o] = cdj
                        rad = rad + cdj * cdj
                    rb[3, sl16o] = rad

            for k in (0, 1):
                pltpu.make_async_copy(b_hbm.at[pl.ds(0, CH)],
                                      ab.at[pl.ds(k * CH, CH)],
                                      gsems[p]).wait()
            wj, jj = loc(j)
            base = wj * EPT + jj * CG2
            pltpu.async_copy(ab, out_hbm.at[pl.ds(base, CG2)], wsems[p])
            pltpu.async_copy(rb, rad_hbm.at[:, pl.ds(base, CG2)], rsems[p])
        return carry

    lax.fori_loop(0, n_my // 2, outer, 0)
    pltpu.make_async_copy(abuf1, out_hbm.at[pl.ds(0, CG2)], wsem1).wait()
    pltpu.make_async_copy(rbuf1, rad_hbm.at[:, pl.ds(0, CG2)], rsem1).wait()


def _sc_gather(a_tab, b_tab, xflat, rc4):
    mesh = plsc.VectorSubcoreMesh(core_axis_name="c", subcore_axis_name="s")
    return pl.kernel(
        _gather_body,
        out_type=[
            jax.ShapeDtypeStruct((E_PAD, D), _F32),
            jax.ShapeDtypeStruct((4, E_PAD), _F32),
        ],
        mesh=mesh,
        scratch_types=[
            pltpu.VMEM((4 * N_PAD,), _F32),
            pltpu.VMEM((2, 2, CH), jnp.int32),
            pltpu.VMEM((2, 2, CH), jnp.int32),
            pltpu.VMEM((CG2, D), _F32),
            pltpu.VMEM((CG2, D), _F32),
            pltpu.VMEM((4, CG2), _F32),
            pltpu.VMEM((4, CG2), _F32),
            pltpu.SemaphoreType.DMA,
            pltpu.SemaphoreType.DMA,
            pltpu.SemaphoreType.DMA,
            pltpu.SemaphoreType.DMA,
            pltpu.SemaphoreType.DMA,
            pltpu.SemaphoreType.DMA,
        ],
        compiler_params=pltpu.CompilerParams(needs_layout_passes=False),
    )(a_tab, b_tab, xflat, rc4)


def _edge_body(g_ref, rad_ref, ea_ref, wea_ref, wradr_ref, we2_ref, be2_ref,
               wc1_ref, bc1_ref, wc2r_ref, bc2_ref, mij_ref, fs_ref):
    radt = rad_ref[...].reshape(1, EB)
    rterm = lax.dot_general(radt, wradr_ref[...], (((0,), (0,)), ((), ())),
                            preferred_element_type=_F32)
    pre = g_ref[...] + rterm + jnp.dot(ea_ref[...], wea_ref[...],
                                       preferred_element_type=_F32)
    m = jax.nn.silu(pre)
    mij = jax.nn.silu(jnp.dot(m, we2_ref[...], preferred_element_type=_F32)
                      + be2_ref[...])
    t = jax.nn.silu(jnp.dot(mij, wc1_ref[...], preferred_element_type=_F32)
                    + bc1_ref[...])
    fst = lax.dot_general(wc2r_ref[...], t, (((1,), (1,)), ((), ())),
                          preferred_element_type=_F32) + bc2_ref[...]
    mij_ref[...] = mij
    fs_ref[...] = fst.reshape(1, 1, EB)


def _edge_mlp(g2, rad3, ea_pad, wea, wradr, we2, be2, wc1, bc1, wc2r, bc2):
    wspec = pl.BlockSpec((D, D), lambda i: (0, 0))
    bspec = pl.BlockSpec((1, D), lambda i: (0, 0))
    return pl.pallas_call(
        _edge_body,
        grid=(NE_BLK,),
        in_specs=[
            pl.BlockSpec((EB, D), lambda i: (i, 0)),
            pl.BlockSpec((1, 1, EB), lambda i: (i, 0, 0)),
            pl.BlockSpec((EB, ED), lambda i: (i, 0)),
            pl.BlockSpec((ED, D), lambda i: (0, 0)),
            bspec, wspec, bspec, wspec, bspec, bspec,
            pl.BlockSpec((1, 1), lambda i: (0, 0)),
        ],
        out_specs=[
            pl.BlockSpec((EB, D), lambda i: (i, 0)),
            pl.BlockSpec((1, 1, EB), lambda i: (i, 0, 0)),
        ],
        out_shape=[
            jax.ShapeDtypeStruct((E_PAD, D), _F32),
            jax.ShapeDtypeStruct((NE_BLK, 1, EB), _F32),
        ],
    )(g2, rad3, ea_pad, wea, wradr, we2, be2, wc1, bc1, wc2r, bc2)


def _scatter_body(m_hbm, fs_hbm, cd_hbm, rc_hbm, outh_hbm, outx_hbm,
                  idxb, xidx, fsbuf, cdbuf, mbuf, fvbuf, acc_h, acc_x):
    c = lax.axis_index("c")
    s = lax.axis_index("s")
    wid = s * 2 + c

    def zrow(i, carry):
        for jb in range(8):
            mbuf[i, pl.ds(jb * 16, 16)] = jnp.zeros((16,), _F32)
            fvbuf[i, pl.ds(jb * 16, 16)] = jnp.zeros((16,), _F32)
        return carry

    lax.fori_loop(0, CH, zrow, 0)
    for k in range(N_PAD // 16 // CH):
        pltpu.sync_copy(mbuf, acc_h.at[pl.ds(s * (N_PAD // 16) + k * CH, CH)])

    @pl.when(s < NXR // 32)
    def _():
        pltpu.sync_copy(fvbuf.at[pl.ds(0, 32)], acc_x.at[pl.ds(s * 32, 32)])

    plsc.subcore_barrier()
    lanes = lax.iota(jnp.int32, 16)

    def chunk(j, carry):
        base = wid * EPT + j * CH
        pltpu.sync_copy(rc_hbm.at[wid].at[j], idxb)
        pltpu.sync_copy(m_hbm.at[pl.ds(base, CH)], mbuf)
        pltpu.sync_copy(mbuf, acc_h.at[idxb.at[0]], add=True)
        pltpu.sync_copy(fs_hbm.at[pl.ds(base, CH)], fsbuf)
        pltpu.sync_copy(cd_hbm.at[:3, pl.ds(base, CH)], cdbuf)
        for g in range(CH // 16):
            sl16 = pl.ds(g * 16, 16)
            rv = idxb[0, sl16]
            fsv = fsbuf[sl16]
            evec = lanes + (g * 16)
            lane0 = (rv & 31) * 4
            for j2 in range(3):
                plsc.store_scatter(fvbuf, [evec, lane0 + j2],
                                   cdbuf[j2, sl16] * fsv)
            xidx[sl16] = lax.shift_right_logical(rv, 5)
        pltpu.sync_copy(fvbuf, acc_x.at[xidx], add=True)
        zv = jnp.zeros((16,), _F32)
        for g in range(CH // 16):
            sl16 = pl.ds(g * 16, 16)
            lane0 = (idxb[0, sl16] & 31) * 4
            evec = lanes + (g * 16)
            for j2 in range(3):
                plsc.store_scatter(fvbuf, [evec, lane0 + j2], zv)
        return carry

    lax.fori_loop(0, NCH, chunk, 0)
    plsc.subcore_barrier()
    pltpu.sync_copy(acc_h.at[pl.ds(s * (N_PAD // 16), N_PAD // 16)],
                    outh_hbm.at[c].at[pl.ds(s * (N_PAD // 16), N_PAD // 16)])

    @pl.when(s < NXR // 32)
    def _():
        pltpu.sync_copy(acc_x.at[pl.ds(s * 32, 32)],
                        outx_hbm.at[c].at[pl.ds(s * 32, 32)])


def _sc_scatter(mij, fs1, cdh, rc4):
    mesh = plsc.VectorSubcoreMesh(core_axis_name="c", subcore_axis_name="s")
    return pl.kernel(
        _scatter_body,
        out_type=[
            jax.ShapeDtypeStruct((2, N_PAD, D), _F32),
            jax.ShapeDtypeStruct((2, NXR, D), _F32),
        ],
        mesh=mesh,
        scratch_types=[
            pltpu.VMEM((2, CH), jnp.int32),
            pltpu.VMEM((CH,), jnp.int32),
            pltpu.VMEM((CH,), _F32),
            pltpu.VMEM((3, CH), _F32),
            pltpu.VMEM((CH, D), _F32),
            pltpu.VMEM((CH, D), _F32),
            pltpu.VMEM_SHARED((N_PAD, D), _F32),
            pltpu.VMEM_SHARED((NXR, D), _F32),
        ],
        compiler_params=pltpu.CompilerParams(needs_layout_passes=False),
    )(mij, fs1, cdh, rc4)


def _node_body(h_ref, x4_ref, p0_ref, p1_ref, q0_ref, q1_ref,
               wn1h_ref, wn1m_ref, bn1_ref, wn2_ref, bn2_ref,
               hn_ref, xn_ref):
    hb = h_ref[...]
    mi = p0_ref[...] + p1_ref[...]
    xu = q0_ref[...] + q1_ref[...]
    u = jax.nn.silu(jnp.dot(hb, wn1h_ref[...], preferred_element_type=_F32)
                    + jnp.dot(mi, wn1m_ref[...], preferred_element_type=_F32)
                    + bn1_ref[...])
    hn_ref[...] = hb + jnp.dot(u, wn2_ref[...], preferred_element_type=_F32) \
        + bn2_ref[...]
    xn_ref[...] = x4_ref[...] + xu


def _node_mlp(h, x4, p0, p1, q0, q1, wn1h, wn1m, bn1, wn2, bn2):
    wspec = pl.BlockSpec((D, D), lambda i: (0, 0))
    bspec = pl.BlockSpec((1, D), lambda i: (0, 0))
    return pl.pallas_call(
        _node_body,
        grid=(N // NB,),
        in_specs=[
            pl.BlockSpec((NB, D), lambda i: (i, 0)),
            pl.BlockSpec((NB, 4), lambda i: (i, 0)),
            pl.BlockSpec((NB, D), lambda i: (i, 0)),
            pl.BlockSpec((NB, D), lambda i: (i, 0)),
            pl.BlockSpec((NB, 4), lambda i: (i, 0)),
            pl.BlockSpec((NB, 4), lambda i: (i, 0)),
            wspec, wspec, bspec, wspec, bspec,
        ],
        out_specs=[
            pl.BlockSpec((NB, D), lambda i: (i, 0)),
            pl.BlockSpec((NB, 4), lambda i: (i, 0)),
        ],
        out_shape=[
            jax.ShapeDtypeStruct((N, D), _F32),
            jax.ShapeDtypeStruct((N, 4), _F32),
        ],
    )(h, x4, p0, p1, q0, q1, wn1h, wn1m, bn1, wn2, bn2)


def kernel(h, x, edge_attr, We1, be1, We2, be2, Wc1, bc1, Wc2, bc2,
           Wn1, bn1, Wn2, bn2, edge_index):
    row = edge_index[0]
    col = edge_index[1]
    x4 = jnp.pad(x, ((0, 0), (0, 1)))
    xflat = jnp.pad(x, ((0, N_PAD - N), (0, 1))).reshape(-1)
    h_pad = jnp.pad(h, ((0, N_PAD - N), (0, 0)))

    wa = We1[:D]
    wb = We1[D:2 * D]
    wradr = We1[2 * D].reshape(1, D)
    wea = We1[2 * D + 1:]
    be1r = be1.reshape(1, D)
    be2r = be2.reshape(1, D)
    bc1r = bc1.reshape(1, D)
    wc2r = Wc2.reshape(1, D)
    bc2r = bc2.reshape(1, 1)
    wn1h = Wn1[:D]
    wn1m = Wn1[D:]
    bn1r = bn1.reshape(1, D)
    bn2r = bn2.reshape(1, D)

    pad = E_PAD - E
    row_p = jnp.concatenate([row, jnp.full((pad,), N, jnp.int32)])
    col_p = jnp.concatenate([col, jnp.full((pad,), N, jnp.int32)])
    rc4 = jnp.stack([row_p.reshape(NT, NCH, CH),
                     col_p.reshape(NT, NCH, CH)], axis=2)
    ea_pad = jnp.concatenate([edge_attr, jnp.zeros((pad, ED), _F32)])

    a_tab, b_tab = _build_tables(h_pad, wa, wb, be1r)
    g2, cdh = _sc_gather(a_tab, b_tab, xflat, rc4)
    rad3 = cdh[3].reshape(NE_BLK, 1, EB)
    mij, fs2 = _edge_mlp(g2, rad3, ea_pad, wea, wradr, We2, be2r,
                         Wc1, bc1r, wc2r, bc2r)
    parts_h, parts_x = _sc_scatter(mij, fs2.reshape(E_PAD), cdh, rc4)
    q0 = parts_x[0].reshape(N_PAD, 4)
    q1 = parts_x[1].reshape(N_PAD, 4)
    h_new, xn4 = _node_mlp(h, x4, parts_h[0], parts_h[1], q0, q1,
                           wn1h, wn1m, bn1r, Wn2, bn2r)
    return (h_new, xn4[:, :3])


# speedup vs baseline: 1.2577x; 1.2577x over previous
import jax
import jax.numpy as jnp
from jax import lax
from jax.experimental import pallas as pl
from jax.experimental.pallas import tpu as pltpu
from jax.experimental.pallas import tpu_sc as plsc

N = 10000
E = 320000
D = 128
ED = 16

NT = 32
CH = 128
NH = 2
E_PAD = 327680
E_H = E_PAD // NH
EPT = E_H // NT
NCH = EPT // CH
N_PAD = 10240
NXR = N_PAD // 32

C_SLOW = 1
CG2 = 256
NCHG = EPT // CG2
KG_SLOW = 14

EB = 512
NE_BLK = E_H // EB
NB = 400
NBP = 512

_F32 = jnp.float32


def _pre_body(h_ref, wa_ref, wb_ref, be1_ref, a_ref, b_ref):
    hb = h_ref[...]
    a_ref[...] = jnp.dot(hb, wa_ref[...], preferred_element_type=_F32) \
        + be1_ref[...]
    b_ref[...] = jnp.dot(hb, wb_ref[...], preferred_element_type=_F32)


def _build_tables(h_pad, wa, wb, be1):
    return pl.pallas_call(
        _pre_body,
        grid=(N_PAD // NBP,),
        in_specs=[
            pl.BlockSpec((NBP, D), lambda i: (i, 0)),
            pl.BlockSpec((D, D), lambda i: (0, 0)),
            pl.BlockSpec((D, D), lambda i: (0, 0)),
            pl.BlockSpec((1, D), lambda i: (0, 0)),
        ],
        out_specs=[
            pl.BlockSpec((NBP, D), lambda i: (i, 0)),
            pl.BlockSpec((NBP, D), lambda i: (i, 0)),
        ],
        out_shape=[
            jax.ShapeDtypeStruct((N_PAD, D), _F32),
            jax.ShapeDtypeStruct((N_PAD, D), _F32),
        ],
    )(h_pad, wa, wb, be1)


def _gather_body(a_hbm, b_hbm, xf_hbm, rc_hbm, out_hbm, rad_hbm,
                 xref, idx0, idx1, abuf0, abuf1, rbuf0, rbuf1,
                 gsem0, gsem1, wsem0, wsem1, rsem0, rsem1):
    c = lax.axis_index("c")
    s = lax.axis_index("s")
    wid = s * 2 + c
    n_my = jnp.where(c == C_SLOW, KG_SLOW, 2 * NCHG - KG_SLOW)
    partner = s * 2 + C_SLOW

    def loc(j):
        sel = j < NCHG
        w_j = jnp.where(sel, wid, partner)
        jj = jnp.where(sel, j, j - (NCHG - KG_SLOW))
        return w_j, jj

    pltpu.sync_copy(xf_hbm, xref)
    idxb = (idx0, idx1)
    abufs = (abuf0, abuf1)
    rbufs = (rbuf0, rbuf1)
    gsems = (gsem0, gsem1)
    wsems = (wsem0, wsem1)
    rsems = (rsem0, rsem1)

    def load_idx(j, q):
        wn, jn = loc(j)
        pltpu.sync_copy(rc_hbm.at[wn].at[pl.ds(jn * 2, 2)], idxb[q])

    def issue_a(q):
        for k in (0, 1):
            pltpu.async_copy(a_hbm.at[idxb[q].at[k].at[0]],
                             abufs[q].at[pl.ds(k * CH, CH)], gsems[q])

    load_idx(0, 0)
    issue_a(0)

    def outer(jo, carry):
        j2 = jo * 2
        for p in (0, 1):
            j = j2 + p
            q = 1 - p
            ab = abufs[p]
            rb = rbufs[p]

            @pl.when(j >= 1)
            def _():
                pltpu.make_async_copy(ab, out_hbm.at[pl.ds(0, CG2)],
                                      wsems[q]).wait()
                pltpu.make_async_copy(rb, rad_hbm.at[:, pl.ds(0, CG2)],
                                      rsems[q]).wait()

            @pl.when(j + 1 < n_my)
            def _():
                load_idx(j + 1, q)
                issue_a(q)

            for k in (0, 1):
                pltpu.make_async_copy(a_hbm.at[pl.ds(0, CH)],
                                      ab.at[pl.ds(k * CH, CH)],
                                      gsems[p]).wait()
            for k in (0, 1):
                pltpu.async_copy(b_hbm.at[idxb[p].at[k].at[1]],
                                 ab.at[pl.ds(k * CH, CH)], gsems[p], add=True)

            for k in (0, 1):
                for g in range(CH // 16):
                    sl16 = pl.ds(g * 16, 16)
                    sl16o = pl.ds(k * CH + g * 16, 16)
                    rv4 = idxb[p][k, 0, sl16] * 4
                    cv4 = idxb[p][k, 1, sl16] * 4
                    rad = jnp.zeros((16,), _F32)
                    for j2c in range(3):
                        cdj = plsc.load_gather(xref, [rv4 + j2c]) \
                            - plsc.load_gather(xref, [cv4 + j2c])
                        rb[j2c, sl16o] = cdj
                        rad = rad + cdj * cdj
                    rb[3, sl16o] = rad

            for k in (0, 1):
                pltpu.make_async_copy(b_hbm.at[pl.ds(0, CH)],
                                      ab.at[pl.ds(k * CH, CH)],
                                      gsems[p]).wait()
            wj, jj = loc(j)
            base = wj * EPT + jj * CG2
            pltpu.async_copy(ab, out_hbm.at[pl.ds(base, CG2)], wsems[p])
            pltpu.async_copy(rb, rad_hbm.at[:, pl.ds(base, CG2)], rsems[p])
        return carry

    lax.fori_loop(0, n_my // 2, outer, 0)
    pltpu.make_async_copy(abuf1, out_hbm.at[pl.ds(0, CG2)], wsem1).wait()
    pltpu.make_async_copy(rbuf1, rad_hbm.at[:, pl.ds(0, CG2)], rsem1).wait()


def _sc_gather(a_tab, b_tab, xflat, rc4):
    mesh = plsc.VectorSubcoreMesh(core_axis_name="c", subcore_axis_name="s")
    return pl.kernel(
        _gather_body,
        out_type=[
            jax.ShapeDtypeStruct((E_H, D), _F32),
            jax.ShapeDtypeStruct((4, E_H), _F32),
        ],
        mesh=mesh,
        scratch_types=[
            pltpu.VMEM((4 * N_PAD,), _F32),
            pltpu.VMEM((2, 2, CH), jnp.int32),
            pltpu.VMEM((2, 2, CH), jnp.int32),
            pltpu.VMEM((CG2, D), _F32),
            pltpu.VMEM((CG2, D), _F32),
            pltpu.VMEM((4, CG2), _F32),
            pltpu.VMEM((4, CG2), _F32),
            pltpu.SemaphoreType.DMA,
            pltpu.SemaphoreType.DMA,
            pltpu.SemaphoreType.DMA,
            pltpu.SemaphoreType.DMA,
            pltpu.SemaphoreType.DMA,
            pltpu.SemaphoreType.DMA,
        ],
        compiler_params=pltpu.CompilerParams(needs_layout_passes=False),
    )(a_tab, b_tab, xflat, rc4)


def _edge_body(g_ref, rad_ref, ea_ref, wea_ref, wradr_ref, we2_ref, be2_ref,
               wc1_ref, bc1_ref, wc2r_ref, bc2_ref, mij_ref, fs_ref):
    radt = rad_ref[...].reshape(1, EB)
    rterm = lax.dot_general(radt, wradr_ref[...], (((0,), (0,)), ((), ())),
                            preferred_element_type=_F32)
    pre = g_ref[...] + rterm + jnp.dot(ea_ref[...], wea_ref[...],
                                       preferred_element_type=_F32)
    m = jax.nn.silu(pre)
    mij = jax.nn.silu(jnp.dot(m, we2_ref[...], preferred_element_type=_F32)
                      + be2_ref[...])
    t = jax.nn.silu(jnp.dot(mij, wc1_ref[...], preferred_element_type=_F32)
                    + bc1_ref[...])
    fst = lax.dot_general(wc2r_ref[...], t, (((1,), (1,)), ((), ())),
                          preferred_element_type=_F32) + bc2_ref[...]
    mij_ref[...] = mij
    fs_ref[...] = fst.reshape(1, 1, EB)


def _edge_mlp(g2, rad3, ea_pad, wea, wradr, we2, be2, wc1, bc1, wc2r, bc2):
    wspec = pl.BlockSpec((D, D), lambda i: (0, 0))
    bspec = pl.BlockSpec((1, D), lambda i: (0, 0))
    return pl.pallas_call(
        _edge_body,
        grid=(NE_BLK,),
        in_specs=[
            pl.BlockSpec((EB, D), lambda i: (i, 0)),
            pl.BlockSpec((1, 1, EB), lambda i: (i, 0, 0)),
            pl.BlockSpec((EB, ED), lambda i: (i, 0)),
            pl.BlockSpec((ED, D), lambda i: (0, 0)),
            bspec, wspec, bspec, wspec, bspec, bspec,
            pl.BlockSpec((1, 1), lambda i: (0, 0)),
        ],
        out_specs=[
            pl.BlockSpec((EB, D), lambda i: (i, 0)),
            pl.BlockSpec((1, 1, EB), lambda i: (i, 0, 0)),
        ],
        out_shape=[
            jax.ShapeDtypeStruct((E_H, D), _F32),
            jax.ShapeDtypeStruct((NE_BLK, 1, EB), _F32),
        ],
    )(g2, rad3, ea_pad, wea, wradr, we2, be2, wc1, bc1, wc2r, bc2)


def _scatter_body(m_hbm, fs_hbm, cd_hbm, rc_hbm, outh_hbm, outx_hbm,
                  idxb, xidx, fsbuf, cdbuf, mbuf, fvbuf, acc_h, acc_x):
    c = lax.axis_index("c")
    s = lax.axis_index("s")
    wid = s * 2 + c

    def zrow(i, carry):
        for jb in range(8):
            mbuf[i, pl.ds(jb * 16, 16)] = jnp.zeros((16,), _F32)
            fvbuf[i, pl.ds(jb * 16, 16)] = jnp.zeros((16,), _F32)
        return carry

    lax.fori_loop(0, CH, zrow, 0)
    for k in range(N_PAD // 16 // CH):
        pltpu.sync_copy(mbuf, acc_h.at[pl.ds(s * (N_PAD // 16) + k * CH, CH)])

    @pl.when(s < NXR // 32)
    def _():
        pltpu.sync_copy(fvbuf.at[pl.ds(0, 32)], acc_x.at[pl.ds(s * 32, 32)])

    plsc.subcore_barrier()
    lanes = lax.iota(jnp.int32, 16)

    def chunk(j, carry):
        base = wid * EPT + j * CH
        pltpu.sync_copy(rc_hbm.at[wid].at[j], idxb)
        pltpu.sync_copy(m_hbm.at[pl.ds(base, CH)], mbuf)
        pltpu.sync_copy(mbuf, acc_h.at[idxb.at[0]], add=True)
        pltpu.sync_copy(fs_hbm.at[pl.ds(base, CH)], fsbuf)
        pltpu.sync_copy(cd_hbm.at[:3, pl.ds(base, CH)], cdbuf)
        for g in range(CH // 16):
            sl16 = pl.ds(g * 16, 16)
            rv = idxb[0, sl16]
            fsv = fsbuf[sl16]
            evec = lanes + (g * 16)
            lane0 = (rv & 31) * 4
            for j2 in range(3):
                plsc.store_scatter(fvbuf, [evec, lane0 + j2],
                                   cdbuf[j2, sl16] * fsv)
            xidx[sl16] = lax.shift_right_logical(rv, 5)
        pltpu.sync_copy(fvbuf, acc_x.at[xidx], add=True)
        zv = jnp.zeros((16,), _F32)
        for g in range(CH // 16):
            sl16 = pl.ds(g * 16, 16)
            lane0 = (idxb[0, sl16] & 31) * 4
            evec = lanes + (g * 16)
            for j2 in range(3):
                plsc.store_scatter(fvbuf, [evec, lane0 + j2], zv)
        return carry

    lax.fori_loop(0, NCH, chunk, 0)
    plsc.subcore_barrier()
    pltpu.sync_copy(acc_h.at[pl.ds(s * (N_PAD // 16), N_PAD // 16)],
                    outh_hbm.at[c].at[pl.ds(s * (N_PAD // 16), N_PAD // 16)])

    @pl.when(s < NXR // 32)
    def _():
        pltpu.sync_copy(acc_x.at[pl.ds(s * 32, 32)],
                        outx_hbm.at[c].at[pl.ds(s * 32, 32)])


def _sc_scatter(mij, fs1, cdh, rc4):
    mesh = plsc.VectorSubcoreMesh(core_axis_name="c", subcore_axis_name="s")
    return pl.kernel(
        _scatter_body,
        out_type=[
            jax.ShapeDtypeStruct((2, N_PAD, D), _F32),
            jax.ShapeDtypeStruct((2, NXR, D), _F32),
        ],
        mesh=mesh,
        scratch_types=[
            pltpu.VMEM((2, CH), jnp.int32),
            pltpu.VMEM((CH,), jnp.int32),
            pltpu.VMEM((CH,), _F32),
            pltpu.VMEM((3, CH), _F32),
            pltpu.VMEM((CH, D), _F32),
            pltpu.VMEM((CH, D), _F32),
            pltpu.VMEM_SHARED((N_PAD, D), _F32),
            pltpu.VMEM_SHARED((NXR, D), _F32),
        ],
        compiler_params=pltpu.CompilerParams(needs_layout_passes=False),
    )(mij, fs1, cdh, rc4)


def _node_body(h_ref, x4_ref, p0_ref, p1_ref, p2_ref, p3_ref,
               q0_ref, q1_ref, q2_ref, q3_ref,
               wn1h_ref, wn1m_ref, bn1_ref, wn2_ref, bn2_ref,
               hn_ref, xn_ref):
    hb = h_ref[...]
    mi = p0_ref[...] + p1_ref[...] + p2_ref[...] + p3_ref[...]
    xu = q0_ref[...] + q1_ref[...] + q2_ref[...] + q3_ref[...]
    u = jax.nn.silu(jnp.dot(hb, wn1h_ref[...], preferred_element_type=_F32)
                    + jnp.dot(mi, wn1m_ref[...], preferred_element_type=_F32)
                    + bn1_ref[...])
    hn_ref[...] = hb + jnp.dot(u, wn2_ref[...], preferred_element_type=_F32) \
        + bn2_ref[...]
    xn_ref[...] = x4_ref[...] + xu


def _node_mlp(h, x4, ps, qs, wn1h, wn1m, bn1, wn2, bn2):
    wspec = pl.BlockSpec((D, D), lambda i: (0, 0))
    bspec = pl.BlockSpec((1, D), lambda i: (0, 0))
    return pl.pallas_call(
        _node_body,
        grid=(N // NB,),
        in_specs=[
            pl.BlockSpec((NB, D), lambda i: (i, 0)),
            pl.BlockSpec((NB, 4), lambda i: (i, 0)),
        ] + [pl.BlockSpec((NB, D), lambda i: (i, 0))] * 4
          + [pl.BlockSpec((NB, 4), lambda i: (i, 0))] * 4
          + [wspec, wspec, bspec, wspec, bspec],
        out_specs=[
            pl.BlockSpec((NB, D), lambda i: (i, 0)),
            pl.BlockSpec((NB, 4), lambda i: (i, 0)),
        ],
        out_shape=[
            jax.ShapeDtypeStruct((N, D), _F32),
            jax.ShapeDtypeStruct((N, 4), _F32),
        ],
    )(h, x4, *ps, *qs, wn1h, wn1m, bn1, wn2, bn2)


def kernel(h, x, edge_attr, We1, be1, We2, be2, Wc1, bc1, Wc2, bc2,
           Wn1, bn1, Wn2, bn2, edge_index):
    row = edge_index[0]
    col = edge_index[1]
    x4 = jnp.pad(x, ((0, 0), (0, 1)))
    xflat = jnp.pad(x, ((0, N_PAD - N), (0, 1))).reshape(-1)
    h_pad = jnp.pad(h, ((0, N_PAD - N), (0, 0)))

    wa = We1[:D]
    wb = We1[D:2 * D]
    wradr = We1[2 * D].reshape(1, D)
    wea = We1[2 * D + 1:]
    be1r = be1.reshape(1, D)
    be2r = be2.reshape(1, D)
    bc1r = bc1.reshape(1, D)
    wc2r = Wc2.reshape(1, D)
    bc2r = bc2.reshape(1, 1)
    wn1h = Wn1[:D]
    wn1m = Wn1[D:]
    bn1r = bn1.reshape(1, D)
    bn2r = bn2.reshape(1, D)

    pad = E_PAD - E
    row_p = jnp.concatenate([row, jnp.full((pad,), N, jnp.int32)])
    col_p = jnp.concatenate([col, jnp.full((pad,), N, jnp.int32)])
    ea_pad = jnp.concatenate([edge_attr, jnp.zeros((pad, ED), _F32)])

    a_tab, b_tab = _build_tables(h_pad, wa, wb, be1r)
    ps, qs = [], []
    for h2 in range(NH):
        sl = slice(h2 * E_H, (h2 + 1) * E_H)
        rc4 = jnp.stack([row_p[sl].reshape(NT, NCH, CH),
                         col_p[sl].reshape(NT, NCH, CH)], axis=2)
        g2, cdh = _sc_gather(a_tab, b_tab, xflat, rc4)
        rad3 = cdh[3].reshape(NE_BLK, 1, EB)
        mij, fs2 = _edge_mlp(g2, rad3, ea_pad[sl], wea, wradr, We2, be2r,
                             Wc1, bc1r, wc2r, bc2r)
        parts_h, parts_x = _sc_scatter(mij, fs2.reshape(E_H), cdh, rc4)
        ps += [parts_h[0], parts_h[1]]
        qs += [parts_x[0].reshape(N_PAD, 4), parts_x[1].reshape(N_PAD, 4)]
    h_new, xn4 = _node_mlp(h, x4, ps, qs, wn1h, wn1m, bn1r, Wn2, bn2r)
    return (h_new, xn4[:, :3])
